# baseline (speedup 1.0000x reference)
import functools

import jax
import jax.numpy as jnp
from jax import lax
from jax.experimental import pallas as pl
from jax.experimental.pallas import tpu as pltpu
from jax.experimental.pallas import tpu_sc as plsc

N = 10000
D = 128
E = 320000

NCORE = 2
NSUB = 16
NW = NCORE * NSUB
K = 64
C = 160
EPAD = NW * C * K
NPAD = 10240
RPT = NPAD // NSUB

_F32 = jnp.float32


def _sc_body(with_deg, *refs):
    if with_deg:
        (x_hbm, comb_hbm, zrows_hbm, zvec_hbm, out_hbm, deg_hbm,
         comb_v, sa0, da0, sa1, da1, sa2, da2,
         b0, b1, b2, ones_v, g0, g1, g2, s0, s1, s2,
         acc, dacc) = refs
    else:
        (x_hbm, comb_hbm, zrows_hbm, out_hbm,
         comb_v, sa0, da0, sa1, da1, sa2, da2,
         b0, b1, b2, ones_v, g0, g1, g2, s0, s1, s2,
         acc) = refs
        deg_hbm = dacc = zvec_hbm = None
    bufs = (b0, b1, b2)
    gsems = (g0, g1, g2)
    ssems = (s0, s1, s2)
    stage = ((sa0, da0), (sa1, da1), (sa2, da2))

    c = lax.axis_index("c")
    s = lax.axis_index("s")
    wid = c * NSUB + s

    pltpu.sync_copy(comb_hbm.at[wid], comb_v)
    pltpu.sync_copy(zrows_hbm, acc.at[pl.ds(s * RPT, RPT)])
    if with_deg:
        pltpu.sync_copy(zvec_hbm.at[pl.ds(s * RPT, RPT)],
                        dacc.at[pl.ds(s * RPT, RPT)])

    def _ones(i, carry):
        ones_v[pl.ds(16 * i, 16)] = jnp.ones((16,), _F32)
        return carry
    lax.fori_loop(0, K // 16, _ones, 0)
    plsc.subcore_barrier()

    def _stage_idx(k, p):
        sv, dv = stage[p]
        for u in range(K // 16):
            v = comb_v[k, pl.ds(16 * u, 16)]
            sv[pl.ds(16 * u, 16)] = v & 0xFFFF
            dv[pl.ds(16 * u, 16)] = lax.shift_right_logical(v, 16)

    def _fire_gather(p):
        pltpu.async_copy(x_hbm.at[stage[p][0]], bufs[p], gsems[p])

    def _wait_gather(p):
        pltpu.make_async_copy(x_hbm.at[stage[p][0]], bufs[p],
                              gsems[p]).wait()

    def _fire_scatter(p):
        pltpu.async_copy(bufs[p], acc.at[stage[p][1]], ssems[p], add=True)
        if with_deg:
            pltpu.async_copy(ones_v, dacc.at[stage[p][1]], ssems[p],
                             add=True)

    def _wait_scatter(p):
        pltpu.make_async_copy(x_hbm.at[pl.ds(0, K)], bufs[p],
                              ssems[p]).wait()
        if with_deg:
            pltpu.make_async_copy(zvec_hbm.at[pl.ds(0, K)], ones_v,
                                  ssems[p]).wait()

    def _step(k, p, m, fire_next):
        _wait_gather(p)
        _fire_scatter(p)
        _wait_scatter(m)
        if fire_next:
            _stage_idx(k + 2, m)
            _fire_gather(m)

    _stage_idx(0, 0)
    _fire_gather(0)
    _stage_idx(1, 1)
    _fire_gather(1)
    _wait_gather(0)
    _fire_scatter(0)
    _stage_idx(2, 2)
    _fire_gather(2)
    _step(1, 1, 0, True)

    def _group(i, carry):
        k0 = 2 + 3 * i
        _step(k0, 2, 1, True)
        _step(k0 + 1, 0, 2, True)
        _step(k0 + 2, 1, 0, True)
        return carry
    lax.fori_loop(0, (C - 4) // 3, _group, 0)
    _step(C - 2, 2, 1, False)
    _step(C - 1, 0, 2, False)
    _wait_scatter(0)

    plsc.subcore_barrier()

    pltpu.sync_copy(acc.at[pl.ds(s * RPT, RPT)],
                    out_hbm.at[c, pl.ds(s * RPT, RPT)])
    if with_deg:
        pltpu.sync_copy(dacc.at[pl.ds(s * RPT, RPT)],
                        deg_hbm.at[pl.ds(c * NPAD + s * RPT, RPT)])


def _make_sc_agg(with_deg):
    mesh = plsc.VectorSubcoreMesh(core_axis_name="c", subcore_axis_name="s")
    out_type = [jax.ShapeDtypeStruct((NCORE, NPAD, D), _F32)]
    if with_deg:
        out_type.append(jax.ShapeDtypeStruct((NCORE * NPAD,), _F32))
    scratch = (
        [pltpu.VMEM((C, K), jnp.int32)]
        + [pltpu.VMEM((K,), jnp.int32)] * 6
        + [pltpu.VMEM((K, D), _F32)] * 3
        + [pltpu.VMEM((K,), _F32)]
        + [pltpu.SemaphoreType.DMA] * 6
        + [pltpu.VMEM_SHARED((NPAD, D), _F32)]
    )
    if with_deg:
        scratch.append(pltpu.VMEM_SHARED((NPAD,), _F32))
    return pl.kernel(
        functools.partial(_sc_body, with_deg),
        out_type,
        mesh=mesh,
        scratch_types=scratch,
        name="sc_edge_agg" + ("_deg" if with_deg else ""),
    )


_sc_agg_deg = _make_sc_agg(True)
_sc_agg = _make_sc_agg(False)


BN = 1000


def _enc_body(h_ref, w_ref, b_ref, o_ref):
    o_ref[...] = h_ref[...] @ w_ref[...] + b_ref[...]


def _encoder(h, w, b2d):
    grid = N // BN
    return pl.pallas_call(
        _enc_body,
        grid=(grid,),
        in_specs=[
            pl.BlockSpec((BN, D), lambda i: (i, 0)),
            pl.BlockSpec((D, D), lambda i: (0, 0)),
            pl.BlockSpec((1, D), lambda i: (0, 0)),
        ],
        out_specs=pl.BlockSpec((BN, D), lambda i: (i, 0)),
        out_shape=jax.ShapeDtypeStruct((N, D), _F32),
    )(h, w, b2d)


def _scale_from_deg(deg_ref):
    dsum = deg_ref[:, 0:1] + deg_ref[:, 1:2]
    dsum = jnp.maximum(dsum, 1.0)
    r = lax.rsqrt(dsum)
    return r * r * r


def _layer_body(x_ref, p_ref, deg_ref, w1a_ref, w1b_ref, b1_ref, w2_ref,
                b2_ref, o_ref):
    x = x_ref[...]
    p = p_ref[...]
    agg = (p[0] + p[1]) * _scale_from_deg(deg_ref)
    z = jnp.maximum(x @ w1a_ref[...] + agg @ w1b_ref[...] + b1_ref[...], 0.0)
    o_ref[...] = x + z @ w2_ref[...] + b2_ref[...]


def _layer_pool_body(x_ref, p_ref, deg_ref, w1a_ref, w1b_ref, b1_ref, w2_ref,
                     b2_ref, o_ref, pool_ref):
    x = x_ref[...]
    p = p_ref[...]
    agg = (p[0] + p[1]) * _scale_from_deg(deg_ref)
    z = jnp.maximum(x @ w1a_ref[...] + agg @ w1b_ref[...] + b1_ref[...], 0.0)
    xo = x + z @ w2_ref[...] + b2_ref[...]
    o_ref[...] = xo

    @pl.when(pl.program_id(0) == 0)
    def _():
        pool_ref[...] = jnp.zeros((1, D), _F32)
    pool_ref[...] += jnp.sum(xo, axis=0, keepdims=True)


def _layer_specs():
    return [
        pl.BlockSpec((BN, D), lambda i: (i, 0)),
        pl.BlockSpec((NCORE, BN, D), lambda i: (0, i, 0)),
        pl.BlockSpec((BN, NCORE), lambda i: (i, 0)),
        pl.BlockSpec((D, D), lambda i: (0, 0)),
        pl.BlockSpec((D, D), lambda i: (0, 0)),
        pl.BlockSpec((1, D), lambda i: (0, 0)),
        pl.BlockSpec((D, D), lambda i: (0, 0)),
        pl.BlockSpec((1, D), lambda i: (0, 0)),
    ]


def _layer(x, p, degT, w1a, w1b, b1, w2, b2):
    return pl.pallas_call(
        _layer_body,
        grid=(N // BN,),
        in_specs=_layer_specs(),
        out_specs=pl.BlockSpec((BN, D), lambda i: (i, 0)),
        out_shape=jax.ShapeDtypeStruct((N, D), _F32),
    )(x, p, degT, w1a, w1b, b1, w2, b2)


def _layer_pool(x, p, degT, w1a, w1b, b1, w2, b2):
    return pl.pallas_call(
        _layer_pool_body,
        grid=(N // BN,),
        in_specs=_layer_specs(),
        out_specs=[
            pl.BlockSpec((BN, D), lambda i: (i, 0)),
            pl.BlockSpec((1, D), lambda i: (0, 0)),
        ],
        out_shape=[
            jax.ShapeDtypeStruct((N, D), _F32),
            jax.ShapeDtypeStruct((1, D), _F32),
        ],
    )(x, p, degT, w1a, w1b, b1, w2, b2)


def _readout_body(pool_ref, w1_ref, b1_ref, w2_ref, b2_ref, w3_ref, b3_ref,
                  o_ref):
    hg = pool_ref[...] * (1.0 / N)
    r = jnp.maximum(hg @ w1_ref[...] + b1_ref[...], 0.0)
    r = jnp.maximum(r @ w2_ref[...] + b2_ref[...], 0.0)
    o_ref[...] = r @ w3_ref[...] + b3_ref[...]


def _readout(pool, w1p, b1p, w2p, b2p, w3p, b3p):
    return pl.pallas_call(
        _readout_body,
        grid=(1,),
        in_specs=[pl.BlockSpec((1, D), lambda i: (0, 0)),
                  pl.BlockSpec((D, D), lambda i: (0, 0)),
                  pl.BlockSpec((1, D), lambda i: (0, 0)),
                  pl.BlockSpec((D, D), lambda i: (0, 0)),
                  pl.BlockSpec((1, D), lambda i: (0, 0)),
                  pl.BlockSpec((D, D), lambda i: (0, 0)),
                  pl.BlockSpec((1, D), lambda i: (0, 0))],
        out_specs=pl.BlockSpec((1, D), lambda i: (0, 0)),
        out_shape=jax.ShapeDtypeStruct((1, D), _F32),
    )(pool, w1p, b1p, w2p, b2p, w3p, b3p)


def _pad_mat(w, rows, cols):
    return jnp.zeros((rows, cols), _F32).at[:w.shape[0], :w.shape[1]].set(w)


def _pad_vec(b, cols):
    return jnp.zeros((1, cols), _F32).at[0, :b.shape[0]].set(b)


def kernel(h, edge_index, e, W_enc, b_enc, W1_0, b1_0, W2_0, b2_0, W1_1, b1_1,
           W2_1, b2_1, W1_2, b1_2, W2_2, b2_2, Wr1, br1, Wr2, br2, Wr3, br3):
    del e

    src = edge_index[0]
    dst = edge_index[1]
    pad = EPAD - E
    ar = jnp.arange(pad, dtype=jnp.int32)
    pad_src = (ar * 37) % N
    pad_dst = N + ar % (NPAD - N)
    srcp = jnp.concatenate([src, pad_src])
    dstp = jnp.concatenate([dst, pad_dst])
    comb = (srcp | (dstp << 16)).reshape(NW, C, K)
    zrows = jnp.zeros((RPT, D), _F32)
    zvec = jnp.zeros((NPAD,), _F32)

    x = _encoder(h, W_enc, b_enc.reshape(1, D))

    p0, deg_flat = _sc_agg_deg(x, comb, zrows, zvec)
    degT = deg_flat.reshape(NCORE, NPAD).T

    hid = W1_0.shape[1]
    x = _layer(x, p0, degT, W1_0[:hid], W1_0[hid:], b1_0.reshape(1, D),
               W2_0, b2_0.reshape(1, D))
    (p1,) = _sc_agg(x, comb, zrows)
    x = _layer(x, p1, degT, W1_1[:hid], W1_1[hid:], b1_1.reshape(1, D),
               W2_1, b2_1.reshape(1, D))
    (p2,) = _sc_agg(x, comb, zrows)
    x, pool = _layer_pool(x, p2, degT, W1_2[:hid], W1_2[hid:],
                          b1_2.reshape(1, D), W2_2, b2_2.reshape(1, D))

    out = _readout(pool,
                   _pad_mat(Wr1, D, D), _pad_vec(br1, D),
                   _pad_mat(Wr2, D, D), _pad_vec(br2, D),
                   _pad_mat(Wr3, D, D), _pad_vec(br3, D))
    return out[:, :Wr3.shape[1]]


# speedup vs baseline: 1.0597x; 1.0597x over previous
import functools

import jax
import jax.numpy as jnp
from jax import lax
from jax.experimental import pallas as pl
from jax.experimental.pallas import tpu as pltpu
from jax.experimental.pallas import tpu_sc as plsc

N = 10000
D = 128
E = 320000

NCORE = 2
NSUB = 16
NW = NCORE * NSUB
K = 64
C = 160
EPAD = NW * C * K
NPAD = 10240
RPT = NPAD // NSUB

_F32 = jnp.float32


def _sc_body(with_deg, *refs):
    if with_deg:
        (x_hbm, comb_hbm, zrows_hbm, zvec_hbm, out_hbm, deg_hbm,
         comb_v, sa0, da0, sa1, da1, sa2, da2,
         b0, b1, b2, ones_v, g0, g1, g2,
         acc, dacc) = refs
    else:
        (x_hbm, comb_hbm, zrows_hbm, out_hbm,
         comb_v, sa0, da0, sa1, da1, sa2, da2,
         b0, b1, b2, ones_v, g0, g1, g2,
         acc) = refs
        deg_hbm = dacc = zvec_hbm = None
    bufs = (b0, b1, b2)
    gsems = (g0, g1, g2)
    stage = ((sa0, da0), (sa1, da1), (sa2, da2))

    c = lax.axis_index("c")
    s = lax.axis_index("s")
    wid = c * NSUB + s

    pltpu.sync_copy(comb_hbm.at[wid], comb_v)
    pltpu.sync_copy(zrows_hbm, acc.at[pl.ds(s * RPT, RPT)])
    if with_deg:
        pltpu.sync_copy(zvec_hbm.at[pl.ds(s * RPT, RPT)],
                        dacc.at[pl.ds(s * RPT, RPT)])

    def _ones(i, carry):
        ones_v[pl.ds(16 * i, 16)] = jnp.ones((16,), _F32)
        return carry
    lax.fori_loop(0, K // 16, _ones, 0)
    plsc.subcore_barrier()

    def _stage_idx(k, p):
        sv, dv = stage[p]
        for u in range(K // 16):
            v = comb_v[k, pl.ds(16 * u, 16)]
            sv[pl.ds(16 * u, 16)] = v & 0xFFFF
            dv[pl.ds(16 * u, 16)] = lax.shift_right_logical(v, 16)

    def _fire_gather(p):
        pltpu.async_copy(x_hbm.at[stage[p][0]], bufs[p], gsems[p])

    def _wait_gather(p):
        pltpu.make_async_copy(x_hbm.at[stage[p][0]], bufs[p],
                              gsems[p]).wait()

    def _step(k, p, fire_next):
        _wait_gather(p)
        pltpu.sync_copy(bufs[p], acc.at[stage[p][1]], add=True)
        if with_deg:
            pltpu.sync_copy(ones_v, dacc.at[stage[p][1]], add=True)
        if fire_next:
            _stage_idx(k + 3, p)
            _fire_gather(p)

    for p in range(3):
        _stage_idx(p, p)
        _fire_gather(p)

    def _group(i, carry):
        k0 = 3 * i
        _step(k0, 0, True)
        _step(k0 + 1, 1, True)
        _step(k0 + 2, 2, True)
        return carry
    lax.fori_loop(0, (C - 4) // 3, _group, 0)
    _step(C - 4, 0, True)
    _step(C - 3, 1, False)
    _step(C - 2, 2, False)
    _step(C - 1, 0, False)

    plsc.subcore_barrier()

    pltpu.sync_copy(acc.at[pl.ds(s * RPT, RPT)],
                    out_hbm.at[c, pl.ds(s * RPT, RPT)])
    if with_deg:
        pltpu.sync_copy(dacc.at[pl.ds(s * RPT, RPT)],
                        deg_hbm.at[pl.ds(c * NPAD + s * RPT, RPT)])


def _make_sc_agg(with_deg):
    mesh = plsc.VectorSubcoreMesh(core_axis_name="c", subcore_axis_name="s")
    out_type = [jax.ShapeDtypeStruct((NCORE, NPAD, D), _F32)]
    if with_deg:
        out_type.append(jax.ShapeDtypeStruct((NCORE * NPAD,), _F32))
    scratch = (
        [pltpu.VMEM((C, K), jnp.int32)]
        + [pltpu.VMEM((K,), jnp.int32)] * 6
        + [pltpu.VMEM((K, D), _F32)] * 3
        + [pltpu.VMEM((K,), _F32)]
        + [pltpu.SemaphoreType.DMA] * 3
        + [pltpu.VMEM_SHARED((NPAD, D), _F32)]
    )
    if with_deg:
        scratch.append(pltpu.VMEM_SHARED((NPAD,), _F32))
    return pl.kernel(
        functools.partial(_sc_body, with_deg),
        out_type,
        mesh=mesh,
        scratch_types=scratch,
        name="sc_edge_agg" + ("_deg" if with_deg else ""),
    )


_sc_agg_deg = _make_sc_agg(True)
_sc_agg = _make_sc_agg(False)


BN = 1000


def _enc_body(h_ref, w_ref, b_ref, o_ref):
    o_ref[...] = h_ref[...] @ w_ref[...] + b_ref[...]


def _encoder(h, w, b2d):
    grid = N // BN
    return pl.pallas_call(
        _enc_body,
        grid=(grid,),
        in_specs=[
            pl.BlockSpec((BN, D), lambda i: (i, 0)),
            pl.BlockSpec((D, D), lambda i: (0, 0)),
            pl.BlockSpec((1, D), lambda i: (0, 0)),
        ],
        out_specs=pl.BlockSpec((BN, D), lambda i: (i, 0)),
        out_shape=jax.ShapeDtypeStruct((N, D), _F32),
    )(h, w, b2d)


def _scale_from_deg(deg_ref):
    dsum = deg_ref[:, 0:1] + deg_ref[:, 1:2]
    dsum = jnp.maximum(dsum, 1.0)
    r = lax.rsqrt(dsum)
    return r * r * r


def _layer_body(x_ref, p_ref, deg_ref, w1a_ref, w1b_ref, b1_ref, w2_ref,
                b2_ref, o_ref):
    x = x_ref[...]
    p = p_ref[...]
    agg = (p[0] + p[1]) * _scale_from_deg(deg_ref)
    z = jnp.maximum(x @ w1a_ref[...] + agg @ w1b_ref[...] + b1_ref[...], 0.0)
    o_ref[...] = x + z @ w2_ref[...] + b2_ref[...]


def _layer_pool_body(x_ref, p_ref, deg_ref, w1a_ref, w1b_ref, b1_ref, w2_ref,
                     b2_ref, wr1_ref, br1_ref, wr2_ref, br2_ref, wr3_ref,
                     br3_ref, o_ref, pool_ref, out_ref):
    x = x_ref[...]
    p = p_ref[...]
    agg = (p[0] + p[1]) * _scale_from_deg(deg_ref)
    z = jnp.maximum(x @ w1a_ref[...] + agg @ w1b_ref[...] + b1_ref[...], 0.0)
    xo = x + z @ w2_ref[...] + b2_ref[...]
    o_ref[...] = xo

    @pl.when(pl.program_id(0) == 0)
    def _():
        pool_ref[...] = jnp.zeros((1, D), _F32)
    pool_ref[...] += jnp.sum(xo, axis=0, keepdims=True)

    @pl.when(pl.program_id(0) == pl.num_programs(0) - 1)
    def _():
        hg = pool_ref[...] * (1.0 / N)
        r = jnp.maximum(hg @ wr1_ref[...] + br1_ref[...], 0.0)
        r = jnp.maximum(r @ wr2_ref[...] + br2_ref[...], 0.0)
        out_ref[...] = r @ wr3_ref[...] + br3_ref[...]


def _layer_specs():
    return [
        pl.BlockSpec((BN, D), lambda i: (i, 0)),
        pl.BlockSpec((NCORE, BN, D), lambda i: (0, i, 0)),
        pl.BlockSpec((BN, NCORE), lambda i: (i, 0)),
        pl.BlockSpec((D, D), lambda i: (0, 0)),
        pl.BlockSpec((D, D), lambda i: (0, 0)),
        pl.BlockSpec((1, D), lambda i: (0, 0)),
        pl.BlockSpec((D, D), lambda i: (0, 0)),
        pl.BlockSpec((1, D), lambda i: (0, 0)),
    ]


def _layer(x, p, degT, w1a, w1b, b1, w2, b2):
    return pl.pallas_call(
        _layer_body,
        grid=(N // BN,),
        in_specs=_layer_specs(),
        out_specs=pl.BlockSpec((BN, D), lambda i: (i, 0)),
        out_shape=jax.ShapeDtypeStruct((N, D), _F32),
    )(x, p, degT, w1a, w1b, b1, w2, b2)


def _layer_pool(x, p, degT, w1a, w1b, b1, w2, b2, wr1, br1, wr2, br2, wr3,
                br3):
    wspec = pl.BlockSpec((D, D), lambda i: (0, 0))
    bspec = pl.BlockSpec((1, D), lambda i: (0, 0))
    return pl.pallas_call(
        _layer_pool_body,
        grid=(N // BN,),
        in_specs=_layer_specs() + [wspec, bspec, wspec, bspec, wspec, bspec],
        out_specs=[
            pl.BlockSpec((BN, D), lambda i: (i, 0)),
            pl.BlockSpec((1, D), lambda i: (0, 0)),
            pl.BlockSpec((1, D), lambda i: (0, 0)),
        ],
        out_shape=[
            jax.ShapeDtypeStruct((N, D), _F32),
            jax.ShapeDtypeStruct((1, D), _F32),
            jax.ShapeDtypeStruct((1, D), _F32),
        ],
    )(x, p, degT, w1a, w1b, b1, w2, b2, wr1, br1, wr2, br2, wr3, br3)


def _pad_mat(w, rows, cols):
    return jnp.zeros((rows, cols), _F32).at[:w.shape[0], :w.shape[1]].set(w)


def _pad_vec(b, cols):
    return jnp.zeros((1, cols), _F32).at[0, :b.shape[0]].set(b)


def kernel(h, edge_index, e, W_enc, b_enc, W1_0, b1_0, W2_0, b2_0, W1_1, b1_1,
           W2_1, b2_1, W1_2, b1_2, W2_2, b2_2, Wr1, br1, Wr2, br2, Wr3, br3):
    del e

    src = edge_index[0]
    dst = edge_index[1]
    pad = EPAD - E
    ar = jnp.arange(pad, dtype=jnp.int32)
    pad_src = (ar * 37) % N
    pad_dst = N + ar % (NPAD - N)
    srcp = jnp.concatenate([src, pad_src])
    dstp = jnp.concatenate([dst, pad_dst])
    comb = (srcp | (dstp << 16)).reshape(NW, C, K)
    zrows = jnp.zeros((RPT, D), _F32)
    zvec = jnp.zeros((NPAD,), _F32)

    x = _encoder(h, W_enc, b_enc.reshape(1, D))

    p0, deg_flat = _sc_agg_deg(x, comb, zrows, zvec)
    degT = deg_flat.reshape(NCORE, NPAD).T

    hid = W1_0.shape[1]
    x = _layer(x, p0, degT, W1_0[:hid], W1_0[hid:], b1_0.reshape(1, D),
               W2_0, b2_0.reshape(1, D))
    (p1,) = _sc_agg(x, comb, zrows)
    x = _layer(x, p1, degT, W1_1[:hid], W1_1[hid:], b1_1.reshape(1, D),
               W2_1, b2_1.reshape(1, D))
    (p2,) = _sc_agg(x, comb, zrows)
    _, _, out = _layer_pool(x, p2, degT, W1_2[:hid], W1_2[hid:],
                            b1_2.reshape(1, D), W2_2, b2_2.reshape(1, D),
                            _pad_mat(Wr1, D, D), _pad_vec(br1, D),
                            _pad_mat(Wr2, D, D), _pad_vec(br2, D),
                            _pad_mat(Wr3, D, D), _pad_vec(br3, D))
    return out[:, :Wr3.shape[1]]


# speedup vs baseline: 1.1200x; 1.0569x over previous
import functools

import jax
import jax.numpy as jnp
from jax import lax
from jax.experimental import pallas as pl
from jax.experimental.pallas import tpu as pltpu
from jax.experimental.pallas import tpu_sc as plsc

N = 10000
D = 128
E = 320000

NCORE = 2
NSUB = 16
NW = NCORE * NSUB
K = 64
C = 160
EPAD = NW * C * K
NPAD = 10240
RPT = NPAD // NSUB

_F32 = jnp.float32


def _sc_body(with_deg, *refs):
    if with_deg:
        (x_hbm, comb_hbm, zrows_hbm, zvec_hbm, out_hbm, deg_hbm,
         comb_v, sa0, da0, sa1, da1, sa2, da2,
         b0, b1, b2, ones_v, g0, g1, g2,
         acc, dacc) = refs
    else:
        (x_hbm, comb_hbm, zrows_hbm, out_hbm,
         comb_v, sa0, da0, sa1, da1, sa2, da2,
         b0, b1, b2, ones_v, g0, g1, g2,
         acc) = refs
        deg_hbm = dacc = zvec_hbm = None
    bufs = (b0, b1, b2)
    gsems = (g0, g1, g2)
    stage = ((sa0, da0), (sa1, da1), (sa2, da2))

    c = lax.axis_index("c")
    s = lax.axis_index("s")
    wid = c * NSUB + s

    pltpu.sync_copy(comb_hbm.at[wid], comb_v)
    pltpu.sync_copy(zrows_hbm, acc.at[pl.ds(s * RPT, RPT)])
    if with_deg:
        pltpu.sync_copy(zvec_hbm.at[pl.ds(s * RPT, RPT)],
                        dacc.at[pl.ds(s * RPT, RPT)])

    def _ones(i, carry):
        ones_v[pl.ds(16 * i, 16)] = jnp.ones((16,), _F32)
        return carry
    lax.fori_loop(0, K // 16, _ones, 0)
    plsc.subcore_barrier()

    def _stage_idx(k, p):
        sv, dv = stage[p]
        for u in range(K // 16):
            v = comb_v[k, pl.ds(16 * u, 16)]
            sv[pl.ds(16 * u, 16)] = v & 0xFFFF
            dv[pl.ds(16 * u, 16)] = lax.shift_right_logical(v, 16)

    def _fire_gather(p):
        pltpu.async_copy(x_hbm.at[stage[p][0]], bufs[p], gsems[p])

    def _wait_gather(p):
        pltpu.make_async_copy(x_hbm.at[stage[p][0]], bufs[p],
                              gsems[p]).wait()

    def _step(k, p, fire_next):
        _wait_gather(p)
        pltpu.sync_copy(bufs[p], acc.at[stage[p][1]], add=True)
        if with_deg:
            pltpu.sync_copy(ones_v, dacc.at[stage[p][1]], add=True)
        if fire_next:
            _stage_idx(k + 3, p)
            _fire_gather(p)

    for p in range(3):
        _stage_idx(p, p)
        _fire_gather(p)

    def _group(i, carry):
        k0 = 3 * i
        _step(k0, 0, True)
        _step(k0 + 1, 1, True)
        _step(k0 + 2, 2, True)
        return carry
    lax.fori_loop(0, (C - 4) // 3, _group, 0)
    _step(C - 4, 0, True)
    _step(C - 3, 1, False)
    _step(C - 2, 2, False)
    _step(C - 1, 0, False)

    plsc.subcore_barrier()

    pltpu.sync_copy(acc.at[pl.ds(s * RPT, RPT)],
                    out_hbm.at[c, pl.ds(s * RPT, RPT)])
    if with_deg:
        pltpu.sync_copy(dacc.at[pl.ds(s * RPT, RPT)],
                        deg_hbm.at[pl.ds(c * NPAD + s * RPT, RPT)])


def _make_sc_agg(with_deg):
    mesh = plsc.VectorSubcoreMesh(core_axis_name="c", subcore_axis_name="s")
    out_type = [jax.ShapeDtypeStruct((NCORE, NPAD, D), _F32)]
    if with_deg:
        out_type.append(jax.ShapeDtypeStruct((NCORE * NPAD,), _F32))
    scratch = (
        [pltpu.VMEM((C, K), jnp.int32)]
        + [pltpu.VMEM((K,), jnp.int32)] * 6
        + [pltpu.VMEM((K, D), _F32)] * 3
        + [pltpu.VMEM((K,), _F32)]
        + [pltpu.SemaphoreType.DMA] * 3
        + [pltpu.VMEM_SHARED((NPAD, D), _F32)]
    )
    if with_deg:
        scratch.append(pltpu.VMEM_SHARED((NPAD,), _F32))
    return pl.kernel(
        functools.partial(_sc_body, with_deg),
        out_type,
        mesh=mesh,
        scratch_types=scratch,
        name="sc_edge_agg" + ("_deg" if with_deg else ""),
    )


_sc_agg_deg = _make_sc_agg(True)
_sc_agg = _make_sc_agg(False)


BN = 2000
GRID = N // BN
EB = E // GRID
EBP = EPAD // GRID
PB = EBP - EB


def _enc_body(h_ref, w_ref, b_ref, ei_ref, o_ref, comb_ref):
    o_ref[...] = h_ref[...] @ w_ref[...] + b_ref[...]
    src = ei_ref[0:1, :]
    dst = ei_ref[1:2, :]
    comb_ref[0, :, :EB] = src | (dst << 16)
    g = (pl.program_id(0) * PB
         + jax.lax.broadcasted_iota(jnp.int32, (1, PB), 1))
    pad_src = (g * 37) % N
    pad_dst = N + g % (NPAD - N)
    comb_ref[0, :, EB:] = pad_src | (pad_dst << 16)


def _encoder(h, w, b2d, edge_index):
    return pl.pallas_call(
        _enc_body,
        grid=(GRID,),
        in_specs=[
            pl.BlockSpec((BN, D), lambda i: (i, 0)),
            pl.BlockSpec((D, D), lambda i: (0, 0)),
            pl.BlockSpec((1, D), lambda i: (0, 0)),
            pl.BlockSpec((2, EB), lambda i: (0, i)),
        ],
        out_specs=[
            pl.BlockSpec((BN, D), lambda i: (i, 0)),
            pl.BlockSpec((1, 1, EBP), lambda i: (i, 0, 0)),
        ],
        out_shape=[
            jax.ShapeDtypeStruct((N, D), _F32),
            jax.ShapeDtypeStruct((GRID, 1, EBP), jnp.int32),
        ],
    )(h, w, b2d, edge_index)


def _scale_from_deg(deg_ref):
    dsum = deg_ref[:, 0:1] + deg_ref[:, 1:2]
    dsum = jnp.maximum(dsum, 1.0)
    r = lax.rsqrt(dsum)
    return r * r * r


def _layer_body(x_ref, p_ref, deg_ref, w1a_ref, w1b_ref, b1_ref, w2_ref,
                b2_ref, o_ref):
    x = x_ref[...]
    p = p_ref[...]
    agg = (p[0] + p[1]) * _scale_from_deg(deg_ref)
    z = jnp.maximum(x @ w1a_ref[...] + agg @ w1b_ref[...] + b1_ref[...], 0.0)
    o_ref[...] = x + z @ w2_ref[...] + b2_ref[...]


def _layer_pool_body(x_ref, p_ref, deg_ref, w1a_ref, w1b_ref, b1_ref, w2_ref,
                     b2_ref, wr1_ref, br1_ref, wr2_ref, br2_ref, wr3_ref,
                     br3_ref, o_ref, pool_ref, out_ref):
    x = x_ref[...]
    p = p_ref[...]
    agg = (p[0] + p[1]) * _scale_from_deg(deg_ref)
    z = jnp.maximum(x @ w1a_ref[...] + agg @ w1b_ref[...] + b1_ref[...], 0.0)
    xo = x + z @ w2_ref[...] + b2_ref[...]
    o_ref[...] = xo

    @pl.when(pl.program_id(0) == 0)
    def _():
        pool_ref[...] = jnp.zeros((1, D), _F32)
    pool_ref[...] += jnp.sum(xo, axis=0, keepdims=True)

    @pl.when(pl.program_id(0) == pl.num_programs(0) - 1)
    def _():
        hg = pool_ref[...] * (1.0 / N)
        r = jnp.maximum(hg @ wr1_ref[...] + br1_ref[...], 0.0)
        r = jnp.maximum(r @ wr2_ref[...] + br2_ref[...], 0.0)
        out_ref[...] = r @ wr3_ref[...] + br3_ref[...]


def _layer_specs():
    return [
        pl.BlockSpec((BN, D), lambda i: (i, 0)),
        pl.BlockSpec((NCORE, BN, D), lambda i: (0, i, 0)),
        pl.BlockSpec((BN, NCORE), lambda i: (i, 0)),
        pl.BlockSpec((D, D), lambda i: (0, 0)),
        pl.BlockSpec((D, D), lambda i: (0, 0)),
        pl.BlockSpec((1, D), lambda i: (0, 0)),
        pl.BlockSpec((D, D), lambda i: (0, 0)),
        pl.BlockSpec((1, D), lambda i: (0, 0)),
    ]


def _layer(x, p, degT, w1a, w1b, b1, w2, b2):
    return pl.pallas_call(
        _layer_body,
        grid=(N // BN,),
        in_specs=_layer_specs(),
        out_specs=pl.BlockSpec((BN, D), lambda i: (i, 0)),
        out_shape=jax.ShapeDtypeStruct((N, D), _F32),
    )(x, p, degT, w1a, w1b, b1, w2, b2)


def _layer_pool(x, p, degT, w1a, w1b, b1, w2, b2, wr1, br1, wr2, br2, wr3,
                br3):
    wspec = pl.BlockSpec((D, D), lambda i: (0, 0))
    bspec = pl.BlockSpec((1, D), lambda i: (0, 0))
    return pl.pallas_call(
        _layer_pool_body,
        grid=(N // BN,),
        in_specs=_layer_specs() + [wspec, bspec, wspec, bspec, wspec, bspec],
        out_specs=[
            pl.BlockSpec((BN, D), lambda i: (i, 0)),
            pl.BlockSpec((1, D), lambda i: (0, 0)),
            pl.BlockSpec((1, D), lambda i: (0, 0)),
        ],
        out_shape=[
            jax.ShapeDtypeStruct((N, D), _F32),
            jax.ShapeDtypeStruct((1, D), _F32),
            jax.ShapeDtypeStruct((1, D), _F32),
        ],
    )(x, p, degT, w1a, w1b, b1, w2, b2, wr1, br1, wr2, br2, wr3, br3)


def _pad_mat(w, rows, cols):
    return jnp.zeros((rows, cols), _F32).at[:w.shape[0], :w.shape[1]].set(w)


def _pad_vec(b, cols):
    return jnp.zeros((1, cols), _F32).at[0, :b.shape[0]].set(b)


def kernel(h, edge_index, e, W_enc, b_enc, W1_0, b1_0, W2_0, b2_0, W1_1, b1_1,
           W2_1, b2_1, W1_2, b1_2, W2_2, b2_2, Wr1, br1, Wr2, br2, Wr3, br3):
    del e

    zrows = jnp.zeros((RPT, D), _F32)
    zvec = jnp.zeros((NPAD,), _F32)

    x, comb_flat = _encoder(h, W_enc, b_enc.reshape(1, D), edge_index)
    comb = comb_flat.reshape(NW, C, K)

    p0, deg_flat = _sc_agg_deg(x, comb, zrows, zvec)
    degT = deg_flat.reshape(NCORE, NPAD).T

    hid = W1_0.shape[1]
    x = _layer(x, p0, degT, W1_0[:hid], W1_0[hid:], b1_0.reshape(1, D),
               W2_0, b2_0.reshape(1, D))
    (p1,) = _sc_agg(x, comb, zrows)
    x = _layer(x, p1, degT, W1_1[:hid], W1_1[hid:], b1_1.reshape(1, D),
               W2_1, b2_1.reshape(1, D))
    (p2,) = _sc_agg(x, comb, zrows)
    _, _, out = _layer_pool(x, p2, degT, W1_2[:hid], W1_2[hid:],
                            b1_2.reshape(1, D), W2_2, b2_2.reshape(1, D),
                            _pad_mat(Wr1, D, D), _pad_vec(br1, D),
                            _pad_mat(Wr2, D, D), _pad_vec(br2, D),
                            _pad_mat(Wr3, D, D), _pad_vec(br3, D))
    return out[:, :Wr3.shape[1]]
